# baseline (speedup 1.0000x reference)
import functools

import jax
import jax.numpy as jnp
from jax import lax
from jax.experimental import pallas as pl
from jax.experimental.pallas import tpu as pltpu
from jax.experimental.pallas import tpu_sc as plsc

_N_TOK = 16384
_D = 2048
_NE = 16
_TOK_BLK = 512

_NW = 32
_TPW = _N_TOK // _NW
_GRP = _TPW // 16


def _logits_body(x_ref, w_ref, b_ref, o_ref):
    o_ref[...] = lax.dot_general(
        w_ref[...], x_ref[...], (((1,), (1,)), ((), ())),
        preferred_element_type=jnp.float32,
    ) + b_ref[...]


def _compute_logits_t(x, gate_w, gate_b):
    nb = _N_TOK // _TOK_BLK
    return pl.pallas_call(
        _logits_body,
        grid=(nb,),
        in_specs=[
            pl.BlockSpec((_TOK_BLK, _D), lambda i: (i, 0)),
            pl.BlockSpec((_NE, _D), lambda i: (0, 0)),
            pl.BlockSpec((_NE, 1), lambda i: (0, 0)),
        ],
        out_specs=pl.BlockSpec((_NE, _TOK_BLK), lambda i: (0, i)),
        out_shape=jax.ShapeDtypeStruct((_NE, _N_TOK), jnp.float32),
    )(x, gate_w, gate_b.reshape(_NE, 1))


def _routing_body(lt_hbm, probs_hbm, idx_hbm, lt_v, probs_v, idx_v):
    c = lax.axis_index("c")
    s = lax.axis_index("s")
    wid = s * 2 + c
    base = wid * _TPW
    pltpu.sync_copy(lt_hbm.at[:, pl.ds(base, _TPW)], lt_v)

    lanes = lax.iota(jnp.int32, 16)
    neg_inf = jnp.full((16,), -jnp.inf, jnp.float32)
    zeros_f = jnp.zeros((16,), jnp.float32)

    def grp(g, carry):
        off = g * 16
        rows = [lt_v[e, pl.ds(off, 16)] for e in range(_NE)]
        m1 = rows[0]
        a1 = jnp.zeros((16,), jnp.int32)
        for e in range(1, _NE):
            upd = rows[e] > m1
            m1 = jnp.where(upd, rows[e], m1)
            a1 = jnp.where(upd, e, a1)
        m2 = neg_inf
        a2 = jnp.zeros((16,), jnp.int32)
        for e in range(_NE):
            v = jnp.where(a1 == e, neg_inf, rows[e])
            upd = v > m2
            m2 = jnp.where(upd, v, m2)
            a2 = jnp.where(upd, e, a2)
        t = jnp.exp(m2 - m1)
        denom = 1.0 + t
        p1 = 1.0 / denom
        p2 = t / denom
        toks = off + lanes
        pbase = toks * _NE
        for e in range(_NE):
            vals = jnp.where(a1 == e, p1, jnp.where(a2 == e, p2, zeros_f))
            plsc.store_scatter(probs_v, [pbase + e], vals)
        ibase = toks * 2
        plsc.store_scatter(idx_v, [ibase], a1)
        plsc.store_scatter(idx_v, [ibase + 1], a2)
        return carry

    lax.fori_loop(0, _GRP, grp, 0)

    pltpu.sync_copy(probs_v, probs_hbm.at[pl.ds(base * _NE, _TPW * _NE)])
    pltpu.sync_copy(idx_v, idx_hbm.at[pl.ds(base * 2, _TPW * 2)])


@functools.cache
def _make_routing():
    return pl.kernel(
        _routing_body,
        mesh=plsc.VectorSubcoreMesh(core_axis_name="c", subcore_axis_name="s"),
        out_type=[
            jax.ShapeDtypeStruct((_N_TOK * _NE,), jnp.float32),
            jax.ShapeDtypeStruct((_N_TOK * 2,), jnp.int32),
        ],
        scratch_types=[
            pltpu.VMEM((_NE, _TPW), jnp.float32),
            pltpu.VMEM((_TPW * _NE,), jnp.float32),
            pltpu.VMEM((_TPW * 2,), jnp.int32),
        ],
        compiler_params=pltpu.CompilerParams(needs_layout_passes=False),
    )


def kernel(x, gate_w, gate_b, noise_w, noise_b):
    logits_t = _compute_logits_t(x, gate_w, gate_b)
    probs_flat, idx_flat = _make_routing()(logits_t)
    return probs_flat.reshape(_N_TOK, _NE), idx_flat.reshape(_N_TOK, 2)


# speedup vs baseline: 1.0618x; 1.0618x over previous
import functools

import jax
import jax.numpy as jnp
from jax import lax
from jax.experimental import pallas as pl
from jax.experimental.pallas import tpu as pltpu
from jax.experimental.pallas import tpu_sc as plsc

_N_TOK = 16384
_D = 2048
_NE = 16
_TOK_BLK = 1024
_CH = 512
_NBUF = 4

_NW = 32
_TPW = _N_TOK // _NW
_GRP = _TPW // 16


def _logits_body(x_hbm, w_ref, b_ref, o_ref, bufs, sems):
    nch = _N_TOK // _CH
    w = w_ref[...]
    b = b_ref[...]

    def start(i, slot):
        pltpu.make_async_copy(
            x_hbm.at[pl.ds(i * _CH, _CH), :], bufs.at[slot], sems.at[slot]
        ).start()

    for i in range(_NBUF):
        start(i, i)
    for i in range(nch):
        slot = i % _NBUF
        pltpu.make_async_copy(
            x_hbm.at[pl.ds(i * _CH, _CH), :], bufs.at[slot], sems.at[slot]
        ).wait()
        o_ref[:, pl.ds(i * _CH, _CH)] = lax.dot_general(
            w, bufs[slot], (((1,), (1,)), ((), ())),
            preferred_element_type=jnp.float32,
        ) + b
        nxt = i + _NBUF
        if nxt < nch:
            start(nxt, slot)


def _compute_logits_t(x, gate_w, gate_b):
    return pl.pallas_call(
        _logits_body,
        in_specs=[
            pl.BlockSpec(memory_space=pl.ANY),
            pl.BlockSpec((_NE, _D), lambda: (0, 0)),
            pl.BlockSpec((_NE, 1), lambda: (0, 0)),
        ],
        out_specs=pl.BlockSpec((_NE, _N_TOK), lambda: (0, 0)),
        out_shape=jax.ShapeDtypeStruct((_NE, _N_TOK), jnp.float32),
        scratch_shapes=[
            pltpu.VMEM((_NBUF, _CH, _D), jnp.float32),
            pltpu.SemaphoreType.DMA((_NBUF,)),
        ],
    )(x, gate_w, gate_b.reshape(_NE, 1))


def _routing_body(lt_hbm, probs_hbm, idx_hbm, lt_v, probs_v, idx_v):
    c = lax.axis_index("c")
    s = lax.axis_index("s")
    wid = s * 2 + c
    base = wid * _TPW
    pltpu.sync_copy(lt_hbm.at[:, pl.ds(base, _TPW)], lt_v)

    lanes = lax.iota(jnp.int32, 16)
    neg_inf = jnp.full((16,), -jnp.inf, jnp.float32)
    zeros_f = jnp.zeros((16,), jnp.float32)

    def grp(g, carry):
        off = g * 16
        rows = [lt_v[e, pl.ds(off, 16)] for e in range(_NE)]
        m1 = rows[0]
        a1 = jnp.zeros((16,), jnp.int32)
        for e in range(1, _NE):
            upd = rows[e] > m1
            m1 = jnp.where(upd, rows[e], m1)
            a1 = jnp.where(upd, e, a1)
        m2 = neg_inf
        a2 = jnp.zeros((16,), jnp.int32)
        for e in range(_NE):
            v = jnp.where(a1 == e, neg_inf, rows[e])
            upd = v > m2
            m2 = jnp.where(upd, v, m2)
            a2 = jnp.where(upd, e, a2)
        t = jnp.exp(m2 - m1)
        denom = 1.0 + t
        p1 = 1.0 / denom
        p2 = t / denom
        toks = off + lanes
        pbase = toks * _NE
        for e in range(_NE):
            vals = jnp.where(a1 == e, p1, jnp.where(a2 == e, p2, zeros_f))
            plsc.store_scatter(probs_v, [pbase + e], vals)
        ibase = toks * 2
        plsc.store_scatter(idx_v, [ibase], a1)
        plsc.store_scatter(idx_v, [ibase + 1], a2)
        return carry

    lax.fori_loop(0, _GRP, grp, 0)

    pltpu.sync_copy(probs_v, probs_hbm.at[pl.ds(base * _NE, _TPW * _NE)])
    pltpu.sync_copy(idx_v, idx_hbm.at[pl.ds(base * 2, _TPW * 2)])


@functools.cache
def _make_routing():
    return pl.kernel(
        _routing_body,
        mesh=plsc.VectorSubcoreMesh(core_axis_name="c", subcore_axis_name="s"),
        out_type=[
            jax.ShapeDtypeStruct((_N_TOK * _NE,), jnp.float32),
            jax.ShapeDtypeStruct((_N_TOK * 2,), jnp.int32),
        ],
        scratch_types=[
            pltpu.VMEM((_NE, _TPW), jnp.float32),
            pltpu.VMEM((_TPW * _NE,), jnp.float32),
            pltpu.VMEM((_TPW * 2,), jnp.int32),
        ],
        compiler_params=pltpu.CompilerParams(needs_layout_passes=False),
    )


def kernel(x, gate_w, gate_b, noise_w, noise_b):
    logits_t = _compute_logits_t(x, gate_w, gate_b)
    probs_flat, idx_flat = _make_routing()(logits_t)
    return probs_flat.reshape(_N_TOK, _NE), idx_flat.reshape(_N_TOK, 2)


# speedup vs baseline: 1.0930x; 1.0293x over previous
import functools

import jax
import jax.numpy as jnp
from jax import lax
from jax.experimental import pallas as pl
from jax.experimental.pallas import tpu as pltpu
from jax.experimental.pallas import tpu_sc as plsc

_N_TOK = 16384
_D = 2048
_NE = 16
_TOK_BLK = 1024

_NW = 32
_TPW = _N_TOK // _NW
_GRP = _TPW // 16


def _logits_body(x_ref, w_ref, b_ref, o_ref):
    o_ref[...] = lax.dot_general(
        w_ref[...], x_ref[...], (((1,), (1,)), ((), ())),
        preferred_element_type=jnp.float32,
    ) + b_ref[...]


def _compute_logits_t(x, gate_w, gate_b):
    nb = _N_TOK // _TOK_BLK
    return pl.pallas_call(
        _logits_body,
        grid=(nb,),
        in_specs=[
            pl.BlockSpec((_TOK_BLK, _D), lambda i: (i, 0)),
            pl.BlockSpec((_NE, _D), lambda i: (0, 0)),
            pl.BlockSpec((_NE, 1), lambda i: (0, 0)),
        ],
        out_specs=pl.BlockSpec((_NE, _TOK_BLK), lambda i: (0, i)),
        out_shape=jax.ShapeDtypeStruct((_NE, _N_TOK), jnp.float32),
    )(x, gate_w, gate_b.reshape(_NE, 1))


def _routing_body(lt_hbm, route_hbm, lt_v, route_v):
    c = lax.axis_index("c")
    s = lax.axis_index("s")
    wid = s * 2 + c
    base = wid * _TPW
    pltpu.sync_copy(lt_hbm.at[:, pl.ds(base, _TPW)], lt_v)

    neg_inf = jnp.full((16,), -jnp.inf, jnp.float32)

    def grp(g, carry):
        off = g * 16
        rows = [lt_v[e, pl.ds(off, 16)] for e in range(_NE)]
        m1 = rows[0]
        a1 = jnp.zeros((16,), jnp.int32)
        for e in range(1, _NE):
            upd = rows[e] > m1
            m1 = jnp.where(upd, rows[e], m1)
            a1 = jnp.where(upd, e, a1)
        m2 = neg_inf
        a2 = jnp.zeros((16,), jnp.int32)
        for e in range(_NE):
            v = jnp.where(a1 == e, neg_inf, rows[e])
            upd = v > m2
            m2 = jnp.where(upd, v, m2)
            a2 = jnp.where(upd, e, a2)
        t = jnp.exp(m2 - m1)
        denom = 1.0 + t
        p1 = 1.0 / denom
        p2 = t / denom
        off_slice = pl.ds(off, 16)
        route_v[0, off_slice] = p1
        route_v[1, off_slice] = p2
        route_v[2, off_slice] = a1.astype(jnp.float32)
        route_v[3, off_slice] = a2.astype(jnp.float32)
        return carry

    lax.fori_loop(0, _GRP, grp, 0)

    for q in range(4):
        pltpu.sync_copy(
            route_v.at[q], route_hbm.at[pl.ds(q * _N_TOK + base, _TPW)])


@functools.cache
def _make_routing():
    return pl.kernel(
        _routing_body,
        mesh=plsc.VectorSubcoreMesh(core_axis_name="c", subcore_axis_name="s"),
        out_type=jax.ShapeDtypeStruct((4 * _N_TOK,), jnp.float32),
        scratch_types=[
            pltpu.VMEM((_NE, _TPW), jnp.float32),
            pltpu.VMEM((4, _TPW), jnp.float32),
        ],
        compiler_params=pltpu.CompilerParams(needs_layout_passes=False),
    )


_EXP_BLK = 2048


def _expand_body(r_ref, probs_ref, idx_ref):
    blk = r_ref[...]
    q_iota = lax.broadcasted_iota(jnp.int32, (4, 1), 0)

    def col(q):
        sel = (q_iota == q).astype(jnp.float32)
        return lax.dot_general(
            blk, sel, (((0,), (0,)), ((), ())),
            preferred_element_type=jnp.float32)

    p1c, p2c, a1c, a2c = col(0), col(1), col(2), col(3)
    a1i = a1c.astype(jnp.int32)
    a2i = a2c.astype(jnp.int32)
    ec = lax.broadcasted_iota(jnp.int32, (_EXP_BLK, _NE), 1)
    probs_ref[...] = (
        jnp.where(ec == a1i, p1c, 0.0) + jnp.where(ec == a2i, p2c, 0.0))
    ic = lax.broadcasted_iota(jnp.int32, (_EXP_BLK, 2), 1)
    idx_ref[...] = jnp.where(ic == 0, a1i, a2i)


def _expand(route):
    nb = _N_TOK // _EXP_BLK
    return pl.pallas_call(
        _expand_body,
        grid=(nb,),
        in_specs=[pl.BlockSpec((4, _EXP_BLK), lambda i: (0, i))],
        out_specs=[
            pl.BlockSpec((_EXP_BLK, _NE), lambda i: (i, 0)),
            pl.BlockSpec((_EXP_BLK, 2), lambda i: (i, 0)),
        ],
        out_shape=[
            jax.ShapeDtypeStruct((_N_TOK, _NE), jnp.float32),
            jax.ShapeDtypeStruct((_N_TOK, 2), jnp.int32),
        ],
    )(route.reshape(4, _N_TOK))


def kernel(x, gate_w, gate_b, noise_w, noise_b):
    logits_t = _compute_logits_t(x, gate_w, gate_b)
    route = _make_routing()(logits_t)
    return _expand(route)


# speedup vs baseline: 1.1245x; 1.0288x over previous
import functools

import jax
import jax.numpy as jnp
from jax import lax
from jax.experimental import pallas as pl
from jax.experimental.pallas import tpu as pltpu
from jax.experimental.pallas import tpu_sc as plsc

_N_TOK = 16384
_D = 2048
_NE = 16
_TOK_BLK = 1024

_NW = 32
_TPW = _N_TOK // _NW
_GRP = _TPW // 16


def _logits_body(x_ref, w_ref, b_ref, o_ref):
    o_ref[...] = lax.dot_general(
        w_ref[...], x_ref[...], (((1,), (1,)), ((), ())),
        preferred_element_type=jnp.float32,
    ) + b_ref[...]


def _compute_logits_t(x, gate_w, gate_b):
    nb = _N_TOK // _TOK_BLK
    return pl.pallas_call(
        _logits_body,
        grid=(nb,),
        in_specs=[
            pl.BlockSpec((_TOK_BLK, _D), lambda i: (i, 0)),
            pl.BlockSpec((_NE, _D), lambda i: (0, 0)),
            pl.BlockSpec((_NE, 1), lambda i: (0, 0)),
        ],
        out_specs=pl.BlockSpec((_NE, _TOK_BLK), lambda i: (0, i)),
        out_shape=jax.ShapeDtypeStruct((_NE, _N_TOK), jnp.float32),
    )(x, gate_w, gate_b.reshape(_NE, 1))


def _routing_body(lt_hbm, route_hbm, lt_v, route_v):
    c = lax.axis_index("c")
    s = lax.axis_index("s")
    wid = s * 2 + c
    base = wid * _TPW
    pltpu.sync_copy(lt_hbm.at[:, pl.ds(base, _TPW)], lt_v)

    neg_inf = jnp.full((16,), -jnp.inf, jnp.float32)

    def grp(g, carry):
        off = g * 16
        rows = [lt_v[e, pl.ds(off, 16)] for e in range(_NE)]
        m1 = rows[0]
        a1 = jnp.zeros((16,), jnp.int32)
        for e in range(1, _NE):
            upd = rows[e] > m1
            m1 = jnp.where(upd, rows[e], m1)
            a1 = jnp.where(upd, e, a1)
        m2 = neg_inf
        a2 = jnp.zeros((16,), jnp.int32)
        for e in range(_NE):
            v = jnp.where(a1 == e, neg_inf, rows[e])
            upd = v > m2
            m2 = jnp.where(upd, v, m2)
            a2 = jnp.where(upd, e, a2)
        t = jnp.exp(m2 - m1)
        denom = 1.0 + t
        p1 = 1.0 / denom
        p2 = t / denom
        off_slice = pl.ds(off, 16)
        route_v[0, off_slice] = p1
        route_v[1, off_slice] = p2
        route_v[2, off_slice] = a1.astype(jnp.float32)
        route_v[3, off_slice] = a2.astype(jnp.float32)
        return carry

    lax.fori_loop(0, _GRP, grp, 0)

    pltpu.sync_copy(route_v, route_hbm.at[:, pl.ds(base, _TPW)])


@functools.cache
def _make_routing():
    return pl.kernel(
        _routing_body,
        mesh=plsc.VectorSubcoreMesh(core_axis_name="c", subcore_axis_name="s"),
        out_type=jax.ShapeDtypeStruct((4, _N_TOK), jnp.float32),
        scratch_types=[
            pltpu.VMEM((_NE, _TPW), jnp.float32),
            pltpu.VMEM((4, _TPW), jnp.float32),
        ],
        compiler_params=pltpu.CompilerParams(needs_layout_passes=False),
    )


_EXP_BLK = 4096


def _expand_body(r_ref, probs_ref, idx_ref):
    blk_t = jnp.transpose(r_ref[...])
    p1c = blk_t[:, 0:1]
    p2c = blk_t[:, 1:2]
    a1c = blk_t[:, 2:3]
    a2c = blk_t[:, 3:4]
    a1i = a1c.astype(jnp.int32)
    a2i = a2c.astype(jnp.int32)
    ec = lax.broadcasted_iota(jnp.int32, (_EXP_BLK, _NE), 1)
    probs_ref[...] = (
        jnp.where(ec == a1i, p1c, 0.0) + jnp.where(ec == a2i, p2c, 0.0))
    ic = lax.broadcasted_iota(jnp.int32, (_EXP_BLK, 2), 1)
    idx_ref[...] = jnp.where(ic == 0, a1i, a2i)


def _expand(route):
    nb = _N_TOK // _EXP_BLK
    return pl.pallas_call(
        _expand_body,
        grid=(nb,),
        in_specs=[pl.BlockSpec((4, _EXP_BLK), lambda i: (0, i))],
        out_specs=[
            pl.BlockSpec((_EXP_BLK, _NE), lambda i: (i, 0)),
            pl.BlockSpec((_EXP_BLK, 2), lambda i: (i, 0)),
        ],
        out_shape=[
            jax.ShapeDtypeStruct((_N_TOK, _NE), jnp.float32),
            jax.ShapeDtypeStruct((_N_TOK, 2), jnp.int32),
        ],
    )(route)


def kernel(x, gate_w, gate_b, noise_w, noise_b):
    logits_t = _compute_logits_t(x, gate_w, gate_b)
    route = _make_routing()(logits_t)
    return _expand(route)


# speedup vs baseline: 1.1246x; 1.0001x over previous
import functools

import jax
import jax.numpy as jnp
from jax import lax
from jax.experimental import pallas as pl
from jax.experimental.pallas import tpu as pltpu
from jax.experimental.pallas import tpu_sc as plsc

_N_TOK = 16384
_D = 2048
_NE = 16
_TOK_BLK = 1024

_NW = 32
_TPW = _N_TOK // _NW
_GRP = _TPW // 16


def _logits_body(x_ref, w_ref, b_ref, o_ref):
    o_ref[...] = lax.dot_general(
        w_ref[...], x_ref[...], (((1,), (1,)), ((), ())),
        preferred_element_type=jnp.float32,
    ) + b_ref[...]


def _compute_logits_t(x, gate_w, gate_b):
    nb = _N_TOK // _TOK_BLK
    return pl.pallas_call(
        _logits_body,
        grid=(nb,),
        in_specs=[
            pl.BlockSpec((_TOK_BLK, _D), lambda i: (i, 0)),
            pl.BlockSpec((_NE, _D), lambda i: (0, 0)),
            pl.BlockSpec((_NE, 1), lambda i: (0, 0)),
        ],
        out_specs=pl.BlockSpec((_NE, _TOK_BLK), lambda i: (0, i)),
        out_shape=jax.ShapeDtypeStruct((_NE, _N_TOK), jnp.float32),
    )(x, gate_w, gate_b.reshape(_NE, 1))


def _routing_body(lt_hbm, route_hbm, lt_v, route_v):
    c = lax.axis_index("c")
    s = lax.axis_index("s")
    wid = s * 2 + c
    base = wid * _TPW
    pltpu.sync_copy(lt_hbm.at[:, pl.ds(base, _TPW)], lt_v)

    neg_inf = jnp.full((16,), -jnp.inf, jnp.float32)

    def grp(g, carry):
        off = g * 16
        rows = [lt_v[e, pl.ds(off, 16)] for e in range(_NE)]
        m1 = rows[0]
        a1 = jnp.zeros((16,), jnp.int32)
        for e in range(1, _NE):
            upd = rows[e] > m1
            m1 = jnp.where(upd, rows[e], m1)
            a1 = jnp.where(upd, e, a1)
        m2 = neg_inf
        a2 = jnp.zeros((16,), jnp.int32)
        for e in range(_NE):
            v = jnp.where(a1 == e, neg_inf, rows[e])
            upd = v > m2
            m2 = jnp.where(upd, v, m2)
            a2 = jnp.where(upd, e, a2)
        t = jnp.exp(m2 - m1)
        denom = 1.0 + t
        p1 = 1.0 / denom
        p2 = t / denom
        off_slice = pl.ds(off, 16)
        route_v[0, off_slice] = p1
        route_v[1, off_slice] = p2
        route_v[2, off_slice] = a1.astype(jnp.float32)
        route_v[3, off_slice] = a2.astype(jnp.float32)
        return carry

    lax.fori_loop(0, _GRP, grp, 0)

    pltpu.sync_copy(route_v, route_hbm.at[pl.ds(0, 4), pl.ds(base, _TPW)])


@functools.cache
def _make_routing():
    return pl.kernel(
        _routing_body,
        mesh=plsc.VectorSubcoreMesh(core_axis_name="c", subcore_axis_name="s"),
        out_type=jax.ShapeDtypeStruct((8, _N_TOK), jnp.float32),
        scratch_types=[
            pltpu.VMEM((_NE, _TPW), jnp.float32),
            pltpu.VMEM((4, _TPW), jnp.float32),
        ],
        compiler_params=pltpu.CompilerParams(needs_layout_passes=False),
    )


_EXP_BLK = 4096


def _expand_body(r_ref, probs_ref, idx_ref):
    blk_t = jnp.transpose(r_ref[...])
    p1c = blk_t[:, 0:1]
    p2c = blk_t[:, 1:2]
    a1c = blk_t[:, 2:3]
    a2c = blk_t[:, 3:4]
    a1i = a1c.astype(jnp.int32)
    a2i = a2c.astype(jnp.int32)
    ec = lax.broadcasted_iota(jnp.int32, (_EXP_BLK, _NE), 1)
    probs_ref[...] = (
        jnp.where(ec == a1i, p1c, 0.0) + jnp.where(ec == a2i, p2c, 0.0))
    ic = lax.broadcasted_iota(jnp.int32, (_EXP_BLK, 2), 1)
    idx_ref[...] = jnp.where(ic == 0, a1i, a2i)


def _expand(route):
    nb = _N_TOK // _EXP_BLK
    return pl.pallas_call(
        _expand_body,
        grid=(nb,),
        in_specs=[pl.BlockSpec((8, _EXP_BLK), lambda i: (0, i))],
        out_specs=[
            pl.BlockSpec((_EXP_BLK, _NE), lambda i: (i, 0)),
            pl.BlockSpec((_EXP_BLK, 2), lambda i: (i, 0)),
        ],
        out_shape=[
            jax.ShapeDtypeStruct((_N_TOK, _NE), jnp.float32),
            jax.ShapeDtypeStruct((_N_TOK, 2), jnp.int32),
        ],
    )(route)


def kernel(x, gate_w, gate_b, noise_w, noise_b):
    logits_t = _compute_logits_t(x, gate_w, gate_b)
    route = _make_routing()(logits_t)
    return _expand(route)


# speedup vs baseline: 1.5398x; 1.3693x over previous
import functools

import jax
import jax.numpy as jnp
from jax import lax
from jax.experimental import pallas as pl
from jax.experimental.pallas import tpu as pltpu
from jax.experimental.pallas import tpu_sc as plsc

_N_TOK = 16384
_D = 2048
_NE = 16
_TOK_BLK = 1024

_NW = 32
_TPW = _N_TOK // _NW
_GRP = _TPW // 16


def _logits_body(x_ref, w_ref, b_ref, o_ref):
    o_ref[...] = lax.dot_general(
        w_ref[...], x_ref[...], (((1,), (1,)), ((), ())),
        preferred_element_type=jnp.float32,
    ) + b_ref[...]


def _compute_logits_t(x, gate_w, gate_b):
    nb = _N_TOK // _TOK_BLK
    return pl.pallas_call(
        _logits_body,
        grid=(nb,),
        in_specs=[
            pl.BlockSpec((_TOK_BLK, _D), lambda i: (i, 0)),
            pl.BlockSpec((_NE, _D), lambda i: (0, 0)),
            pl.BlockSpec((_NE, 1), lambda i: (0, 0)),
        ],
        out_specs=pl.BlockSpec((_NE, _TOK_BLK), lambda i: (0, i)),
        out_shape=jax.ShapeDtypeStruct((_NE, _N_TOK), jnp.float32),
    )(x, gate_w, gate_b.reshape(_NE, 1))


def _routing_body(lt_hbm, probs_t_hbm, idx_t_hbm, lt_v, pt_v, it_v):
    c = lax.axis_index("c")
    s = lax.axis_index("s")
    wid = s * 2 + c
    base = wid * _TPW
    pltpu.sync_copy(lt_hbm.at[:, pl.ds(base, _TPW)], lt_v)

    neg_inf = jnp.full((16,), -jnp.inf, jnp.float32)

    def grp(g, carry):
        off = g * 16
        rows = [lt_v[e, pl.ds(off, 16)] for e in range(_NE)]
        m1 = rows[0]
        a1 = jnp.zeros((16,), jnp.int32)
        for e in range(1, _NE):
            upd = rows[e] > m1
            m1 = jnp.where(upd, rows[e], m1)
            a1 = jnp.where(upd, e, a1)
        m2 = neg_inf
        a2 = jnp.zeros((16,), jnp.int32)
        for e in range(_NE):
            v = jnp.where(a1 == e, neg_inf, rows[e])
            upd = v > m2
            m2 = jnp.where(upd, v, m2)
            a2 = jnp.where(upd, e, a2)
        t = jnp.exp(m2 - m1)
        denom = 1.0 + t
        p1 = 1.0 / denom
        p2 = t / denom
        off_slice = pl.ds(off, 16)
        zeros_f = jnp.zeros((16,), jnp.float32)
        for e in range(_NE):
            pt_v[e, off_slice] = (jnp.where(a1 == e, p1, zeros_f)
                                  + jnp.where(a2 == e, p2, zeros_f))
        it_v[0, off_slice] = a1
        it_v[1, off_slice] = a2
        return carry

    lax.fori_loop(0, _GRP, grp, 0)

    pltpu.sync_copy(pt_v, probs_t_hbm.at[:, pl.ds(base, _TPW)])
    pltpu.sync_copy(it_v, idx_t_hbm.at[:, pl.ds(base, _TPW)])


@functools.cache
def _make_routing():
    return pl.kernel(
        _routing_body,
        mesh=plsc.VectorSubcoreMesh(core_axis_name="c", subcore_axis_name="s"),
        out_type=[
            jax.ShapeDtypeStruct((_NE, _N_TOK), jnp.float32),
            jax.ShapeDtypeStruct((2, _N_TOK), jnp.int32),
        ],
        scratch_types=[
            pltpu.VMEM((_NE, _TPW), jnp.float32),
            pltpu.VMEM((_NE, _TPW), jnp.float32),
            pltpu.VMEM((2, _TPW), jnp.int32),
        ],
        compiler_params=pltpu.CompilerParams(needs_layout_passes=False),
    )


def kernel(x, gate_w, gate_b, noise_w, noise_b):
    logits_t = _compute_logits_t(x, gate_w, gate_b)
    probs_t, idx_t = _make_routing()(logits_t)
    return probs_t.T, idx_t.T
